# Initial kernel scaffold; baseline (speedup 1.0000x reference)
import jax
import jax.numpy as jnp
from jax.experimental import pallas as pl


def kernel(x_user, x_item, params, edge_index_user_item, edge_index_item_user, edge_index):
    raise NotImplementedError("write your pallas kernel here")


# speedup vs baseline: 3.6389x; 3.6389x over previous
import functools

import jax
import jax.numpy as jnp
from jax import lax
from jax.experimental import pallas as pl
from jax.experimental.pallas import tpu as pltpu
from jax.experimental.pallas import tpu_sc as plsc

N_USER = 10000
N_ITEM = 10000
N = 10000
E = 160000
H = 128
EPS = 1e-7

BR = 1000
NT = 16
EP = E // NT
CH = 80
NCH = EP // CH
ACC_N = 10240
RPT = ACC_N // NT
ZB = 80

EP2 = E // (2 * NT)
CH2 = 40
NCH2 = EP2 // CH2


def _pre_body(xs_ref, xd_ref, ws_ref, bs_ref, wd_ref, bd_ref, t_ref,
              a_ref, b_ref, hd_ref):
    hs = jnp.dot(xs_ref[...], ws_ref[...],
                 preferred_element_type=jnp.float32) + bs_ref[...]
    m = jnp.maximum(hs, 0.0) + EPS
    a = jnp.exp(t_ref[...] * m)
    a_ref[...] = a
    b_ref[...] = a * m
    hd_ref[...] = jnp.dot(xd_ref[...], wd_ref[...],
                          preferred_element_type=jnp.float32) + bd_ref[...]


def _pre(x_src, x_dst, p):
    t_row = jnp.broadcast_to(jnp.reshape(p['t'], (1, 1)), (1, H))
    return pl.pallas_call(
        _pre_body,
        grid=(N // BR,),
        in_specs=[
            pl.BlockSpec((BR, H), lambda i: (i, 0)),
            pl.BlockSpec((BR, H), lambda i: (i, 0)),
            pl.BlockSpec((H, H), lambda i: (0, 0)),
            pl.BlockSpec((1, H), lambda i: (0, 0)),
            pl.BlockSpec((H, H), lambda i: (0, 0)),
            pl.BlockSpec((1, H), lambda i: (0, 0)),
            pl.BlockSpec((1, H), lambda i: (0, 0)),
        ],
        out_specs=[
            pl.BlockSpec((BR, H), lambda i: (i, 0)),
            pl.BlockSpec((BR, H), lambda i: (i, 0)),
            pl.BlockSpec((BR, H), lambda i: (i, 0)),
        ],
        out_shape=[jax.ShapeDtypeStruct((N, H), jnp.float32)] * 3,
    )(x_src, x_dst, p['Ws'], p['bs'].reshape(1, H), p['Wd'],
      p['bd'].reshape(1, H), t_row)


def _post_body(den_ref, num_ref, hd_ref, w1_ref, b1_ref, w2_ref, b2_ref,
               out_ref):
    aggr = num_ref[...] / (den_ref[...] + 1e-16)
    h0 = aggr + hd_ref[...]
    h1 = jnp.maximum(
        jnp.dot(h0, w1_ref[...], preferred_element_type=jnp.float32)
        + b1_ref[...], 0.0)
    h2 = jnp.dot(h1, w2_ref[...],
                 preferred_element_type=jnp.float32) + b2_ref[...]
    out_ref[...] = jnp.maximum(h2, 0.0)


def _post(den, num, hd, p):
    return pl.pallas_call(
        _post_body,
        grid=(N // BR,),
        in_specs=[
            pl.BlockSpec((BR, H), lambda i: (i, 0)),
            pl.BlockSpec((BR, H), lambda i: (i, 0)),
            pl.BlockSpec((BR, H), lambda i: (i, 0)),
            pl.BlockSpec((H, 2 * H), lambda i: (0, 0)),
            pl.BlockSpec((1, 2 * H), lambda i: (0, 0)),
            pl.BlockSpec((2 * H, H), lambda i: (0, 0)),
            pl.BlockSpec((1, H), lambda i: (0, 0)),
        ],
        out_specs=pl.BlockSpec((BR, H), lambda i: (i, 0)),
        out_shape=jax.ShapeDtypeStruct((N, H), jnp.float32),
    )(den, num, hd, p['W1'], p['b1'].reshape(1, 2 * H), p['W2'],
      p['b2'].reshape(1, H))


def _zero_buf(buf, rows):
    zv = jnp.zeros((16,), jnp.float32)

    def zrow(i, carry):
        for j in range(H // 16):
            buf[i, pl.ds(j * 16, 16)] = zv
        return carry

    lax.fori_loop(0, rows, zrow, 0)


def _sc_scatter_body(a_hbm, b_hbm, src_hbm, dst_hbm, den_hbm, num_hbm,
                     sidx, didx, rows, zbuf, acc, sem):
    c = lax.axis_index("c")
    s = lax.axis_index("s")

    _zero_buf(zbuf, ZB)
    base_r = s * RPT
    for k in range(RPT // ZB):
        pltpu.sync_copy(zbuf, acc.at[pl.ds(base_r + k * ZB, ZB)])
    plsc.subcore_barrier()

    def run(table_hbm, out_hbm):
        ebase = s * EP

        def step(k, carry):
            off = ebase + k * CH
            pltpu.sync_copy(src_hbm.at[pl.ds(off, CH)], sidx)
            pltpu.sync_copy(dst_hbm.at[pl.ds(off, CH)], didx)
            pltpu.async_copy(table_hbm.at[sidx], rows, sem).wait()
            pltpu.sync_copy(rows, acc.at[didx], add=True)
            return carry

        lax.fori_loop(0, NCH, step, 0)
        plsc.subcore_barrier()
        for k in range(RPT // ZB):
            r0 = base_r + k * ZB

            @pl.when(r0 + ZB <= N)
            def _():
                pltpu.sync_copy(acc.at[pl.ds(r0, ZB)],
                                out_hbm.at[pl.ds(r0, ZB)])

    @pl.when(c == 0)
    def _():
        run(a_hbm, den_hbm)

    @pl.when(c == 1)
    def _():
        run(b_hbm, num_hbm)


@functools.lru_cache(maxsize=None)
def _sc_scatter_kernel():
    return pl.kernel(
        _sc_scatter_body,
        out_type=[jax.ShapeDtypeStruct((N, H), jnp.float32)] * 2,
        mesh=plsc.VectorSubcoreMesh(core_axis_name="c", subcore_axis_name="s"),
        scratch_types=[
            pltpu.VMEM((CH,), jnp.int32),
            pltpu.VMEM((CH,), jnp.int32),
            pltpu.VMEM((CH, H), jnp.float32),
            pltpu.VMEM((ZB, H), jnp.float32),
            pltpu.VMEM_SHARED((ACC_N, H), jnp.float32),
            pltpu.SemaphoreType.DMA,
        ],
    )


def _sc_scatter(a, b, src, dst):
    return _sc_scatter_kernel()(a, b, src, dst)


def _sc_gather2_body(u_hbm, it_hbm, pu_hbm, pi_hbm, ue_hbm, ie_hbm,
                     uidx, iidx, urows, irows, sem0, sem1):
    c = lax.axis_index("c")
    s = lax.axis_index("s")
    wid = s * 2 + c
    ebase = wid * EP2

    def step(k, carry):
        off = ebase + k * CH2
        pltpu.sync_copy(pu_hbm.at[pl.ds(off, CH2)], uidx)
        pltpu.sync_copy(pi_hbm.at[pl.ds(off, CH2)], iidx)
        cp0 = pltpu.async_copy(u_hbm.at[uidx], urows, sem0)
        cp1 = pltpu.async_copy(it_hbm.at[iidx], irows, sem1)
        cp0.wait()
        cp1.wait()
        pltpu.sync_copy(urows, ue_hbm.at[pl.ds(off, CH2)])
        pltpu.sync_copy(irows, ie_hbm.at[pl.ds(off, CH2)])
        return carry

    lax.fori_loop(0, NCH2, step, 0)


@functools.lru_cache(maxsize=None)
def _sc_gather2_kernel():
    return pl.kernel(
        _sc_gather2_body,
        out_type=[jax.ShapeDtypeStruct((E, H), jnp.float32)] * 2,
        mesh=plsc.VectorSubcoreMesh(core_axis_name="c", subcore_axis_name="s"),
        scratch_types=[
            pltpu.VMEM((CH2,), jnp.int32),
            pltpu.VMEM((CH2,), jnp.int32),
            pltpu.VMEM((CH2, H), jnp.float32),
            pltpu.VMEM((CH2, H), jnp.float32),
            pltpu.SemaphoreType.DMA,
            pltpu.SemaphoreType.DMA,
        ],
    )


def _sc_gather2(u, it, pu, pi):
    return _sc_gather2_kernel()(u, it, pu, pi)


def _dot_body(ue_ref, ie_ref, out_ref):
    out_ref[...] = jnp.sum(ue_ref[...] * ie_ref[...], axis=-1, keepdims=True)


def _dot(ue, ie):
    BRD = 2000
    out = pl.pallas_call(
        _dot_body,
        grid=(E // BRD,),
        in_specs=[
            pl.BlockSpec((BRD, H), lambda i: (i, 0)),
            pl.BlockSpec((BRD, H), lambda i: (i, 0)),
        ],
        out_specs=pl.BlockSpec((BRD, 1), lambda i: (i, 0)),
        out_shape=jax.ShapeDtypeStruct((E, 1), jnp.float32),
    )(ue, ie)
    return out.reshape(E)


def kernel(x_user, x_item, params, edge_index_user_item, edge_index_item_user,
           edge_index):
    u, it = x_user, x_item
    s_ui, d_ui = edge_index_user_item[0], edge_index_user_item[1]
    s_iu, d_iu = edge_index_item_user[0], edge_index_item_user[1]
    for l in ('l1', 'l2', 'l3'):
        p_ui = params[l + '_ui']
        p_iu = params[l + '_iu']
        a_ui, b_ui, hd_it = _pre(u, it, p_ui)
        a_iu, b_iu, hd_u = _pre(it, u, p_iu)
        den_it, num_it = _sc_scatter(a_ui, b_ui, s_ui, d_ui)
        den_u, num_u = _sc_scatter(a_iu, b_iu, s_iu, d_iu)
        new_it = _post(den_it, num_it, hd_it, p_ui)
        new_u = _post(den_u, num_u, hd_u, p_iu)
        u, it = new_u, new_it
    ue, ie = _sc_gather2(u, it, edge_index[0], edge_index[1])
    return _dot(ue, ie)
